# baseline (speedup 1.0000x reference)
import functools

import jax
import jax.numpy as jnp
from jax import lax
from jax.experimental import pallas as pl
from jax.experimental.pallas import tpu as pltpu
from jax.experimental.pallas import tpu_sc as plsc

N_NODES = 10000
N_EDGES = 320000
D = 128
NPAD = 10240
NCORES = 2
NSUB = 16
NW = NCORES * NSUB
K = 128
CHUNKS = 80
M_PER_TILE = K * CHUNKS
EPAD = NW * M_PER_TILE
RPT = NPAD // NSUB
HALF_CHUNKS = CHUNKS // 2
HALF_M = K * HALF_CHUNKS

_mesh = plsc.VectorSubcoreMesh(core_axis_name="c", subcore_axis_name="s")


@functools.partial(
    pl.kernel,
    mesh=_mesh,
    out_type=jax.ShapeDtypeStruct((NCORES * NPAD, D), jnp.float32),
    scratch_types=[
        pltpu.VMEM((M_PER_TILE,), jnp.int32),
        pltpu.VMEM((K, D), jnp.float32),
        pltpu.VMEM_SHARED((NPAD, D), jnp.float32),
    ],
)
def _deg_kernel(dst_hbm, ones_hbm, zeros_hbm, out_hbm, didx, ones_v, deg_sp):
    cid = lax.axis_index("c")
    sid = lax.axis_index("s")
    w = cid * NSUB + sid
    pltpu.sync_copy(ones_hbm, ones_v)
    pltpu.sync_copy(dst_hbm.at[pl.ds(w * M_PER_TILE, M_PER_TILE)], didx)
    pltpu.sync_copy(
        zeros_hbm.at[pl.ds(sid * RPT, RPT)],
        deg_sp.at[pl.ds(sid * RPT, RPT)],
    )
    plsc.subcore_barrier()

    def chunk(i, carry):
        pltpu.sync_copy(ones_v, deg_sp.at[didx.at[pl.ds(i * K, K)]], add=True)
        return carry

    lax.fori_loop(0, CHUNKS, chunk, 0)
    plsc.subcore_barrier()
    pltpu.sync_copy(
        deg_sp.at[pl.ds(sid * RPT, RPT)],
        out_hbm.at[pl.ds(cid * NPAD + sid * RPT, RPT)],
    )


_UNROLL = 8
_BLOCKS = CHUNKS // _UNROLL
BLK0 = 12
BLK1 = 2 * _BLOCKS - BLK0

_agg_scratch = (
    [pltpu.VMEM((K,), jnp.int32) for _ in range(2 * _UNROLL)]
    + [pltpu.VMEM((K, D), jnp.float32)]
    + [pltpu.VMEM_SHARED((NPAD, D), jnp.float32)]
    + [pltpu.SemaphoreType.DMA for _ in range(_UNROLL + 1)]
)


@functools.partial(
    pl.kernel,
    mesh=_mesh,
    out_type=jax.ShapeDtypeStruct((NCORES * NPAD, D), jnp.float32),
    scratch_types=_agg_scratch,
)
def _agg_kernel(u_hbm, src_hbm, dst_hbm, zeros_hbm, out_hbm, *scratch):
    sbuf = scratch[:_UNROLL]
    dbuf = scratch[_UNROLL:2 * _UNROLL]
    rows = scratch[2 * _UNROLL]
    agg_sp = scratch[2 * _UNROLL + 1]
    isem = scratch[2 * _UNROLL + 2:2 * _UNROLL + 2 + _UNROLL]
    gsem = scratch[2 * _UNROLL + 2 + _UNROLL]

    cid = lax.axis_index("c")
    sid = lax.axis_index("s")
    base = jnp.where(cid == 0, sid * (BLK0 * _UNROLL * K),
                     NSUB * (BLK0 * _UNROLL * K) + sid * (BLK1 * _UNROLL * K))
    nblocks = jnp.where(cid == 0, BLK0, BLK1)

    pltpu.sync_copy(
        zeros_hbm.at[pl.ds(sid * RPT, RPT)],
        agg_sp.at[pl.ds(sid * RPT, RPT)],
    )
    plsc.subcore_barrier()

    def block(blk, carry):
        bb = base + blk * (_UNROLL * K)
        iobj = []
        for k in range(_UNROLL):
            o1 = pltpu.async_copy(src_hbm.at[pl.ds(bb + k * K, K)], sbuf[k], isem[k])
            o2 = pltpu.async_copy(dst_hbm.at[pl.ds(bb + k * K, K)], dbuf[k], isem[k])
            iobj.append((o1, o2))
        for k in range(_UNROLL):
            iobj[k][0].wait()
            iobj[k][1].wait()
            pltpu.async_copy(u_hbm.at[sbuf[k]], rows, gsem).wait()
            pltpu.sync_copy(rows, agg_sp.at[dbuf[k]], add=True)
        return carry

    lax.fori_loop(0, nblocks, block, 0)
    plsc.subcore_barrier()
    pltpu.sync_copy(
        agg_sp.at[pl.ds(sid * RPT, RPT)],
        out_hbm.at[pl.ds(cid * NPAD + sid * RPT, RPT)],
    )


_B = 1024


def _dinv_from_parts(degp):
    deg = degp[0, :, 0] + degp[1, :, 0] + 1.0
    return lax.rsqrt(deg)


def _tc1_body(x_ref, w_ref, degp_ref, u_ref):
    dinv = _dinv_from_parts(degp_ref[...])
    h = jnp.dot(x_ref[...], w_ref[...], preferred_element_type=jnp.float32)
    u_ref[...] = h * dinv[:, None]


def _tc2_body(agg_ref, u_ref, degp_ref, b_ref, w_ref, u2_ref):
    i = pl.program_id(0)
    dinv = _dinv_from_parts(degp_ref[...])
    s = agg_ref[0] + agg_ref[1] + u_ref[...]
    o1 = jnp.maximum(s * dinv[:, None] + b_ref[...], 0.0)
    u2 = jnp.dot(o1, w_ref[...], preferred_element_type=jnp.float32)
    u2 = u2 * dinv[:, None]
    rows = i * _B + lax.broadcasted_iota(jnp.int32, (_B, D), 0)
    u2_ref[...] = jnp.where(rows < N_NODES, u2, 0.0)


def _tc3_body(agg_ref, u_ref, degp_ref, b_ref, out_ref):
    dinv = _dinv_from_parts(degp_ref[...])
    s = agg_ref[0] + agg_ref[1] + u_ref[...]
    out_ref[...] = jnp.maximum(s * dinv[:, None] + b_ref[...], 0.0)


_row_spec = pl.BlockSpec((_B, D), lambda i: (i, 0))
_degp_spec = pl.BlockSpec((2, _B, D), lambda i: (0, i, 0))
_agg_spec = pl.BlockSpec((2, _B, D), lambda i: (0, i, 0))
_w_spec = pl.BlockSpec((D, D), lambda i: (0, 0))
_b_spec = pl.BlockSpec((1, D), lambda i: (0, 0))
_f32 = jnp.float32


def _tc1(xp, w1p, degp):
    return pl.pallas_call(
        _tc1_body,
        grid=(NPAD // _B,),
        in_specs=[_row_spec, _w_spec, _degp_spec],
        out_specs=_row_spec,
        out_shape=jax.ShapeDtypeStruct((NPAD, D), _f32),
    )(xp, w1p, degp)


def _tc2(agg, u1, degp, b1p, w2p):
    return pl.pallas_call(
        _tc2_body,
        grid=(NPAD // _B,),
        in_specs=[_agg_spec, _row_spec, _degp_spec, _b_spec, _w_spec],
        out_specs=_row_spec,
        out_shape=jax.ShapeDtypeStruct((NPAD, D), _f32),
    )(agg, u1, degp, b1p, w2p)


def _tc3(agg, u2, degp, b2p):
    return pl.pallas_call(
        _tc3_body,
        grid=(NPAD // _B,),
        in_specs=[_agg_spec, _row_spec, _degp_spec, _b_spec],
        out_specs=_row_spec,
        out_shape=jax.ShapeDtypeStruct((NPAD, D), _f32),
    )(agg, u2, degp, b2p)


def kernel(x, edge_index, W1, b1, W2, b2):
    ei = edge_index.astype(jnp.int32)
    npad_e = EPAD - N_EDGES
    fill = jnp.full((npad_e,), N_NODES, jnp.int32)
    src = jnp.concatenate([ei[0], fill])
    dst = jnp.concatenate([ei[1], fill])

    xp = jnp.zeros((NPAD, D), _f32).at[:N_NODES].set(x)
    w1p = jnp.zeros((D, D), _f32).at[:, :126].set(W1)
    b1p = jnp.zeros((1, D), _f32).at[0, :126].set(b1)
    w2p = jnp.zeros((D, D), _f32).at[:126, :126].set(W2)
    b2p = jnp.zeros((1, D), _f32).at[0, :126].set(b2)

    onesD = jnp.ones((K, D), _f32)
    zerosD = jnp.zeros((NPAD, D), _f32)

    degp = _deg_kernel(dst, onesD, zerosD).reshape(NCORES, NPAD, D)
    u1 = _tc1(xp, w1p, degp)
    agg1 = _agg_kernel(u1, src, dst, zerosD).reshape(NCORES, NPAD, D)
    u2 = _tc2(agg1, u1, degp, b1p, w2p)
    agg2 = _agg_kernel(u2, src, dst, zerosD).reshape(NCORES, NPAD, D)
    out = _tc3(agg2, u2, degp, b2p)
    return out[:N_NODES, :126]


# speedup vs baseline: 1.2564x; 1.2564x over previous
import functools

import jax
import jax.numpy as jnp
from jax import lax
from jax.experimental import pallas as pl
from jax.experimental.pallas import tpu as pltpu
from jax.experimental.pallas import tpu_sc as plsc

N_NODES = 10000
N_EDGES = 320000
D = 128
NPAD = 10240
NCORES = 2
NSUB = 16
NW = NCORES * NSUB
K = 128
CHUNKS = 80
M_PER_TILE = K * CHUNKS
EPAD = NW * M_PER_TILE
RPT = NPAD // NSUB
HALF_CHUNKS = CHUNKS // 2
HALF_M = K * HALF_CHUNKS

_mesh = plsc.VectorSubcoreMesh(core_axis_name="c", subcore_axis_name="s")


@functools.partial(
    pl.kernel,
    mesh=_mesh,
    out_type=jax.ShapeDtypeStruct((NCORES * NPAD, D), jnp.float32),
    scratch_types=[
        pltpu.VMEM((M_PER_TILE,), jnp.int32),
        pltpu.VMEM((K, D), jnp.float32),
        pltpu.VMEM_SHARED((NPAD, D), jnp.float32),
    ],
)
def _deg_kernel(dst_hbm, ones_hbm, zeros_hbm, out_hbm, didx, ones_v, deg_sp):
    cid = lax.axis_index("c")
    sid = lax.axis_index("s")
    w = cid * NSUB + sid
    pltpu.sync_copy(ones_hbm, ones_v)
    pltpu.sync_copy(dst_hbm.at[pl.ds(w * M_PER_TILE, M_PER_TILE)], didx)
    pltpu.sync_copy(
        zeros_hbm.at[pl.ds(sid * RPT, RPT)],
        deg_sp.at[pl.ds(sid * RPT, RPT)],
    )
    plsc.subcore_barrier()

    def chunk(i, carry):
        pltpu.sync_copy(ones_v, deg_sp.at[didx.at[pl.ds(i * K, K)]], add=True)
        return carry

    lax.fori_loop(0, CHUNKS, chunk, 0)
    plsc.subcore_barrier()
    pltpu.sync_copy(
        deg_sp.at[pl.ds(sid * RPT, RPT)],
        out_hbm.at[pl.ds(cid * NPAD + sid * RPT, RPT)],
    )


_UNROLL = 8
_BLOCKS = CHUNKS // _UNROLL
BLK0 = 18
BLK1 = 2 * _BLOCKS - BLK0

_agg_scratch = (
    [pltpu.VMEM((K,), jnp.int32) for _ in range(2 * _UNROLL)]
    + [pltpu.VMEM((K, D), jnp.float32)]
    + [pltpu.VMEM_SHARED((NPAD, D), jnp.float32)]
    + [pltpu.SemaphoreType.DMA for _ in range(_UNROLL + 1)]
)


@functools.partial(
    pl.kernel,
    mesh=_mesh,
    out_type=jax.ShapeDtypeStruct((NCORES * NPAD, D), jnp.float32),
    scratch_types=_agg_scratch,
)
def _agg_kernel(u_hbm, src_hbm, dst_hbm, zeros_hbm, out_hbm, *scratch):
    sbuf = scratch[:_UNROLL]
    dbuf = scratch[_UNROLL:2 * _UNROLL]
    rows = scratch[2 * _UNROLL]
    agg_sp = scratch[2 * _UNROLL + 1]
    isem = scratch[2 * _UNROLL + 2:2 * _UNROLL + 2 + _UNROLL]
    gsem = scratch[2 * _UNROLL + 2 + _UNROLL]

    cid = lax.axis_index("c")
    sid = lax.axis_index("s")
    base = jnp.where(cid == 0, sid * (BLK0 * _UNROLL * K),
                     NSUB * (BLK0 * _UNROLL * K) + sid * (BLK1 * _UNROLL * K))
    nblocks = jnp.where(cid == 0, BLK0, BLK1)

    pltpu.sync_copy(
        zeros_hbm.at[pl.ds(sid * RPT, RPT)],
        agg_sp.at[pl.ds(sid * RPT, RPT)],
    )
    plsc.subcore_barrier()

    def block(blk, carry):
        bb = base + blk * (_UNROLL * K)
        iobj = []
        for k in range(_UNROLL):
            o1 = pltpu.async_copy(src_hbm.at[pl.ds(bb + k * K, K)], sbuf[k], isem[k])
            o2 = pltpu.async_copy(dst_hbm.at[pl.ds(bb + k * K, K)], dbuf[k], isem[k])
            iobj.append((o1, o2))
        for k in range(_UNROLL):
            iobj[k][0].wait()
            iobj[k][1].wait()
            pltpu.async_copy(u_hbm.at[sbuf[k]], rows, gsem).wait()
            pltpu.sync_copy(rows, agg_sp.at[dbuf[k]], add=True)
        return carry

    lax.fori_loop(0, nblocks, block, 0)
    plsc.subcore_barrier()
    pltpu.sync_copy(
        agg_sp.at[pl.ds(sid * RPT, RPT)],
        out_hbm.at[pl.ds(cid * NPAD + sid * RPT, RPT)],
    )


_B = 1024


def _dinv_from_parts(degp):
    deg = degp[0, :, 0] + degp[1, :, 0] + 1.0
    return lax.rsqrt(deg)


def _tc1_body(x_ref, w_ref, degp_ref, u_ref):
    dinv = _dinv_from_parts(degp_ref[...])
    h = jnp.dot(x_ref[...], w_ref[...], preferred_element_type=jnp.float32)
    u_ref[...] = h * dinv[:, None]


def _tc2_body(agg_ref, u_ref, degp_ref, b_ref, w_ref, u2_ref):
    i = pl.program_id(0)
    dinv = _dinv_from_parts(degp_ref[...])
    s = agg_ref[0] + agg_ref[1] + u_ref[...]
    o1 = jnp.maximum(s * dinv[:, None] + b_ref[...], 0.0)
    u2 = jnp.dot(o1, w_ref[...], preferred_element_type=jnp.float32)
    u2 = u2 * dinv[:, None]
    rows = i * _B + lax.broadcasted_iota(jnp.int32, (_B, D), 0)
    u2_ref[...] = jnp.where(rows < N_NODES, u2, 0.0)


def _tc3_body(agg_ref, u_ref, degp_ref, b_ref, out_ref):
    dinv = _dinv_from_parts(degp_ref[...])
    s = agg_ref[0] + agg_ref[1] + u_ref[...]
    out_ref[...] = jnp.maximum(s * dinv[:, None] + b_ref[...], 0.0)


_row_spec = pl.BlockSpec((_B, D), lambda i: (i, 0))
_degp_spec = pl.BlockSpec((2, _B, D), lambda i: (0, i, 0))
_agg_spec = pl.BlockSpec((2, _B, D), lambda i: (0, i, 0))
_w_spec = pl.BlockSpec((D, D), lambda i: (0, 0))
_b_spec = pl.BlockSpec((1, D), lambda i: (0, 0))
_f32 = jnp.float32


def _tc1(xp, w1p, degp):
    return pl.pallas_call(
        _tc1_body,
        grid=(NPAD // _B,),
        in_specs=[_row_spec, _w_spec, _degp_spec],
        out_specs=_row_spec,
        out_shape=jax.ShapeDtypeStruct((NPAD, D), _f32),
    )(xp, w1p, degp)


def _tc2(agg, u1, degp, b1p, w2p):
    return pl.pallas_call(
        _tc2_body,
        grid=(NPAD // _B,),
        in_specs=[_agg_spec, _row_spec, _degp_spec, _b_spec, _w_spec],
        out_specs=_row_spec,
        out_shape=jax.ShapeDtypeStruct((NPAD, D), _f32),
    )(agg, u1, degp, b1p, w2p)


def _tc3(agg, u2, degp, b2p):
    return pl.pallas_call(
        _tc3_body,
        grid=(NPAD // _B,),
        in_specs=[_agg_spec, _row_spec, _degp_spec, _b_spec],
        out_specs=_row_spec,
        out_shape=jax.ShapeDtypeStruct((NPAD, D), _f32),
    )(agg, u2, degp, b2p)


def kernel(x, edge_index, W1, b1, W2, b2):
    ei = edge_index.astype(jnp.int32)
    npad_e = EPAD - N_EDGES
    fill = jnp.full((npad_e,), N_NODES, jnp.int32)
    src = jnp.concatenate([ei[0], fill])
    dst = jnp.concatenate([ei[1], fill])

    xp = jnp.zeros((NPAD, D), _f32).at[:N_NODES].set(x)
    w1p = jnp.zeros((D, D), _f32).at[:, :126].set(W1)
    b1p = jnp.zeros((1, D), _f32).at[0, :126].set(b1)
    w2p = jnp.zeros((D, D), _f32).at[:126, :126].set(W2)
    b2p = jnp.zeros((1, D), _f32).at[0, :126].set(b2)

    onesD = jnp.ones((K, D), _f32)
    zerosD = jnp.zeros((NPAD, D), _f32)

    degp = _deg_kernel(dst, onesD, zerosD).reshape(NCORES, NPAD, D)
    u1 = _tc1(xp, w1p, degp)
    agg1 = _agg_kernel(u1, src, dst, zerosD).reshape(NCORES, NPAD, D)
    u2 = _tc2(agg1, u1, degp, b1p, w2p)
    agg2 = _agg_kernel(u2, src, dst, zerosD).reshape(NCORES, NPAD, D)
    out = _tc3(agg2, u2, degp, b2p)
    return out[:N_NODES, :126]
